# baseline (speedup 1.0000x reference)
import jax
import jax.numpy as jnp
from jax import lax
from jax.experimental import pallas as pl
from jax.experimental.pallas import tpu as pltpu
from jax.experimental.pallas import tpu_sc as plsc

_ROWS = 4880
_DIM = 128
_TOTAL = _ROWS * _DIM
_NUM_SUBCORES = 16
_CHUNK = _TOTAL // _NUM_SUBCORES
_PIECE = _CHUNK // 2


def _copy_body(src_hbm, out_hbm, buf0, buf1, rs0, rs1, ws0, ws1):
    wid = lax.axis_index("s")
    base = wid * _CHUNK
    r0 = pltpu.async_copy(src_hbm.at[pl.ds(base, _PIECE)], buf0, rs0)
    r0.wait()
    w0 = pltpu.async_copy(buf0, out_hbm.at[pl.ds(base, _PIECE)], ws0)
    r1 = pltpu.async_copy(src_hbm.at[pl.ds(base + _PIECE, _PIECE)], buf1, rs1)
    r1.wait()
    w1 = pltpu.async_copy(buf1, out_hbm.at[pl.ds(base + _PIECE, _PIECE)], ws1)
    w0.wait()
    w1.wait()


@jax.jit
def kernel(table):
    flat = table.reshape(_TOTAL)
    mesh = plsc.VectorSubcoreMesh(
        core_axis_name="c", subcore_axis_name="s", num_cores=1)
    out = pl.kernel(
        _copy_body,
        out_type=jax.ShapeDtypeStruct((_TOTAL,), jnp.float32),
        scratch_types=[
            pltpu.VMEM((_PIECE,), jnp.float32),
            pltpu.VMEM((_PIECE,), jnp.float32),
            pltpu.SemaphoreType.DMA,
            pltpu.SemaphoreType.DMA,
            pltpu.SemaphoreType.DMA,
            pltpu.SemaphoreType.DMA,
        ],
        mesh=mesh,
    )(flat)
    return out.reshape(_ROWS, _DIM)
